# Initial kernel scaffold; baseline (speedup 1.0000x reference)
import jax
import jax.numpy as jnp
from jax.experimental import pallas as pl


def kernel(x, edge_index, dense_edge_idx, batch, W0, a_src0, a_dst0, b0, W1, a_src1, a_dst1, b1, W2, a_src2, a_dst2, b2, Wf, bf):
    raise NotImplementedError("write your pallas kernel here")


# speedup vs baseline: 34.9982x; 34.9982x over previous
import functools

import jax
import jax.numpy as jnp
from jax import lax
from jax.experimental import pallas as pl
from jax.experimental.pallas import tpu as pltpu
from jax.experimental.pallas import tpu_sc as plsc

N = 10000
E = 320000
G = 64
D = 64
F = 128
NC = 2
NS = 16
L = 16
RB = 80
EPT = E // (NC * NS)
BPT = EPT // RB
ROWS = E // RB
N2 = 10240
NPT = N2 // NS

_EPS = 1e-16


def _sc_body(h_hbm, al_hbm, ar_hbm, src_hbm, dst_hbm, zu_hbm, zd_hbm, cs_hbm,
             u_hbm, den_hbm,
             al_v, ar_v, src_v, dst_v, wblk_v, rows_v, cs_v,
             u_sh, den_sh):
  c = lax.axis_index("c")
  s = lax.axis_index("s")
  tile = c * NS + s
  row0 = tile * BPT
  node0 = s * NPT

  pltpu.sync_copy(al_hbm, al_v)
  pltpu.sync_copy(ar_hbm, ar_v)
  pltpu.sync_copy(src_hbm.at[pl.ds(row0, BPT)], src_v)
  pltpu.sync_copy(dst_hbm.at[pl.ds(row0, BPT)], dst_v)
  pltpu.sync_copy(zu_hbm.at[pl.ds(node0, NPT)], u_sh.at[pl.ds(node0, NPT)])
  pltpu.sync_copy(cs_hbm, cs_v)

  @pl.when(s == 0)
  def _():
    pltpu.sync_copy(zd_hbm, den_sh)

  plsc.subcore_barrier()

  def block(j, carry):
    for g in range(RB // L):
      sl = pl.ds(g * L, L)
      isrc = src_v[j, 0, sl]
      idst = dst_v[j, 0, sl]
      a = plsc.load_gather(al_v, [isrc])
      r = plsc.load_gather(ar_v, [idst])
      e = a + r
      e = jnp.where(e >= 0, e, e * jnp.float32(0.2))
      wblk_v[sl] = jnp.exp(e)

    pltpu.sync_copy(h_hbm.at[src_v.at[j, 0]], rows_v)

    for r in range(RB):
      ws = plsc.load_gather(wblk_v, [cs_v[r]])
      for q in range(D // L):
        sl = pl.ds(q * L, L)
        rows_v[r, sl] = rows_v[r, sl] * ws

    pltpu.sync_copy(rows_v, u_sh.at[dst_v.at[j, 0]], add=True)
    pltpu.sync_copy(wblk_v, den_sh.at[dst_v.at[j, 0]], add=True)
    return carry

  lax.fori_loop(0, BPT, block, 0)
  plsc.subcore_barrier()

  pltpu.sync_copy(u_sh.at[pl.ds(node0, NPT)], u_hbm.at[c, pl.ds(node0, NPT)])

  @pl.when(s == 0)
  def _():
    pltpu.sync_copy(den_sh, den_hbm.at[c])


@functools.partial(jax.jit, static_argnames=())
def _sc_conv(h, al, ar, src2, dst2, zu, zd, cs):
  mesh = plsc.VectorSubcoreMesh(
      core_axis_name="c", subcore_axis_name="s", num_cores=NC, num_subcores=NS
  )
  f32 = jnp.float32
  run = pl.kernel(
      _sc_body,
      out_type=(
          jax.ShapeDtypeStruct((NC, N2, D), f32),
          jax.ShapeDtypeStruct((NC, N2), f32),
      ),
      mesh=mesh,
      compiler_params=pltpu.CompilerParams(
          needs_layout_passes=False, use_tc_tiling_on_sc=False),
      scratch_types=(
          pltpu.VMEM((N,), f32),
          pltpu.VMEM((N,), f32),
          pltpu.VMEM((BPT, 1, RB), jnp.int32),
          pltpu.VMEM((BPT, 1, RB), jnp.int32),
          pltpu.VMEM((RB,), f32),
          pltpu.VMEM((RB, D), f32),
          pltpu.VMEM((RB, L), jnp.int32),
          pltpu.VMEM_SHARED((N2, D), f32),
          pltpu.VMEM_SHARED((N2,), f32),
      ),
  )
  return run(h, al, ar, src2, dst2, zu, zd, cs)


def _tc_layer0_body(x_ref, w_ref, asrc_ref, adst_ref, h_ref, al_ref, ar_ref):
  h = jnp.dot(x_ref[...], w_ref[...], preferred_element_type=jnp.float32)
  h_ref[...] = h
  al_ref[...] = jnp.sum(h * asrc_ref[...], axis=1, keepdims=True)
  ar_ref[...] = jnp.sum(h * adst_ref[...], axis=1, keepdims=True)


def _tc_layer0(x, w, asrc, adst):
  f32 = jnp.float32
  return pl.pallas_call(
      _tc_layer0_body,
      out_shape=(
          jax.ShapeDtypeStruct((N, D), f32),
          jax.ShapeDtypeStruct((N, 1), f32),
          jax.ShapeDtypeStruct((N, 1), f32),
      ),
  )(x, w, asrc, adst)


def _combine(u_ref, den_ref, b_ref, rows=None):
  sl = slice(None) if rows is None else slice(0, rows)
  den = den_ref[0, sl] + den_ref[1, sl] + jnp.float32(_EPS)
  out = (u_ref[0, sl] + u_ref[1, sl]) / den + b_ref[...]
  return jnp.maximum(out, jnp.float32(0.0))

BLK = 2000


def _tc_mid_body(ua_ref, da_ref, ub_ref, db_ref, b_ref, wt_ref, wb_ref,
                 asrc_ref, adst_ref, h_ref, al_ref, ar_ref):
  ra = _combine(ua_ref, da_ref, b_ref)
  rb = _combine(ub_ref, db_ref, b_ref)
  h = (jnp.dot(ra, wt_ref[...], preferred_element_type=jnp.float32)
       + jnp.dot(rb, wb_ref[...], preferred_element_type=jnp.float32))
  h_ref[...] = h
  al_ref[...] = jnp.sum(h * asrc_ref[...], axis=1, keepdims=True)
  ar_ref[...] = jnp.sum(h * adst_ref[...], axis=1, keepdims=True)


def _tc_mid(ua, da, ub, db, b, wt, wb, asrc, adst):
  da = da.reshape(NC, N2, 1)
  db = db.reshape(NC, N2, 1)
  f32 = jnp.float32
  ubs = pl.BlockSpec((NC, BLK, D), lambda i: (0, i, 0))
  dbs = pl.BlockSpec((NC, BLK, 1), lambda i: (0, i, 0))
  full2 = lambda shape: pl.BlockSpec(shape, lambda i: (0, 0))
  return pl.pallas_call(
      _tc_mid_body,
      grid=(N // BLK,),
      in_specs=[ubs, dbs, ubs, dbs,
                full2((1, D)), full2((D, D)), full2((D, D)),
                full2((1, D)), full2((1, D))],
      out_specs=(
          pl.BlockSpec((BLK, D), lambda i: (i, 0)),
          pl.BlockSpec((BLK, 1), lambda i: (i, 0)),
          pl.BlockSpec((BLK, 1), lambda i: (i, 0)),
      ),
      out_shape=(
          jax.ShapeDtypeStruct((N, D), f32),
          jax.ShapeDtypeStruct((N, 1), f32),
          jax.ShapeDtypeStruct((N, 1), f32),
      ),
  )(ua, da, ub, db, b, wt, wb, asrc, adst)


def _tc_final_body(ua_ref, da_ref, ub_ref, db_ref, b_ref, wft_ref, wfb_ref,
                   bf_ref, batch_ref, y_ref):
  ra = _combine(ua_ref, da_ref, b_ref, rows=N)
  rb = _combine(ub_ref, db_ref, b_ref, rows=N)
  z = (jnp.dot(ra, wft_ref[...], preferred_element_type=jnp.float32)
       + jnp.dot(rb, wfb_ref[...], preferred_element_type=jnp.float32))
  gid = lax.broadcasted_iota(jnp.int32, (G, N), 0)
  m = (gid == batch_ref[...]).astype(jnp.float32)
  y = jnp.dot(m, z, preferred_element_type=jnp.float32)
  y_ref[...] = y + bf_ref[...]


def _tc_final(ua, da, ub, db, b, wft, wfb, bf, batch):
  da = da.reshape(NC, N2, 1)
  db = db.reshape(NC, N2, 1)
  return pl.pallas_call(
      _tc_final_body,
      out_shape=jax.ShapeDtypeStruct((G, 1), jnp.float32),
  )(ua, da, ub, db, b, wft, wfb, bf, batch)


def kernel(x, edge_index, dense_edge_idx, batch,
           W0, a_src0, a_dst0, b0,
           W1, a_src1, a_dst1, b1,
           W2, a_src2, a_dst2, b2,
           Wf, bf):
  f32 = jnp.float32
  srcA = edge_index[0].reshape(ROWS, 1, RB)
  dstA = edge_index[1].reshape(ROWS, 1, RB)
  srcB = dense_edge_idx[0].reshape(ROWS, 1, RB)
  dstB = dense_edge_idx[1].reshape(ROWS, 1, RB)
  zu = jnp.zeros((N2, D), f32)
  zd = jnp.zeros((N2,), f32)
  cs = jnp.broadcast_to(jnp.arange(RB, dtype=jnp.int32)[:, None], (RB, L))
  cs = jnp.asarray(cs)
  batch2 = batch.reshape(1, N)

  params = [
      (W0, a_src0, a_dst0, b0),
      (W1, a_src1, a_dst1, b1),
      (W2, a_src2, a_dst2, b2),
  ]

  h, al, ar = _tc_layer0(x, W0, a_src0.reshape(1, D), a_dst0.reshape(1, D))
  for l in (1, 2):
    ua, da = _sc_conv(h, al.reshape(N), ar.reshape(N), srcA, dstA, zu, zd, cs)
    ub, db = _sc_conv(h, al.reshape(N), ar.reshape(N), srcB, dstB, zu, zd, cs)
    W, asrc, adst, _ = params[l]
    b_prev = params[l - 1][3]
    h, al, ar = _tc_mid(
        ua, da, ub, db, b_prev.reshape(1, D),
        W[:D], W[D:], asrc.reshape(1, D), adst.reshape(1, D))

  ua, da = _sc_conv(h, al.reshape(N), ar.reshape(N), srcA, dstA, zu, zd, cs)
  ub, db = _sc_conv(h, al.reshape(N), ar.reshape(N), srcB, dstB, zu, zd, cs)
  y = _tc_final(
      ua, da, ub, db, b2.reshape(1, D),
      Wf[:D], Wf[D:], bf.reshape(1, 1), batch2)
  return y.reshape(G)


# speedup vs baseline: 43.9776x; 1.2566x over previous
import functools

import jax
import jax.numpy as jnp
from jax import lax
from jax.experimental import pallas as pl
from jax.experimental.pallas import tpu as pltpu
from jax.experimental.pallas import tpu_sc as plsc

N = 10000
E = 320000
G = 64
D = 64
F = 128
NC = 2
NS = 16
L = 16
RB = 80
EPT = E // (NC * NS)
BPT = EPT // RB
ROWS = E // RB
N2 = 10240
NPT = N2 // NS

_EPS = 1e-16


def _sc_body(h_hbm, al_hbm, ar_hbm, src_hbm, dst_hbm, zu_hbm, zd_hbm, cs_hbm,
             u_hbm, den_hbm,
             al_v, ar_v, src_v, dst_v, wblk0_v, wblk1_v, rows0_v, rows1_v,
             cs_v, u_sh, den_sh, semg0, semg1, sems0, sems1):
  c = lax.axis_index("c")
  s = lax.axis_index("s")
  tile = c * NS + s
  row0 = tile * BPT
  node0 = s * NPT

  pltpu.sync_copy(al_hbm, al_v)
  pltpu.sync_copy(ar_hbm, ar_v)
  pltpu.sync_copy(src_hbm.at[pl.ds(row0, BPT)], src_v)
  pltpu.sync_copy(dst_hbm.at[pl.ds(row0, BPT)], dst_v)
  pltpu.sync_copy(zu_hbm.at[pl.ds(node0, NPT)], u_sh.at[pl.ds(node0, NPT)])
  pltpu.sync_copy(cs_hbm, cs_v)

  @pl.when(s == 0)
  def _():
    pltpu.sync_copy(zd_hbm, den_sh)

  plsc.subcore_barrier()

  def compute_w(j, wblk):
    for g in range(RB // L):
      sl = pl.ds(g * L, L)
      isrc = src_v[j, 0, sl]
      idst = dst_v[j, 0, sl]
      a = plsc.load_gather(al_v, [isrc])
      r = plsc.load_gather(ar_v, [idst])
      e = a + r
      e = jnp.where(e >= 0, e, e * jnp.float32(0.2))
      wblk[sl] = jnp.exp(e)

  def scale(rows, wblk):
    for r in range(RB):
      ws = plsc.load_gather(wblk, [cs_v[r]])
      for q in range(D // L):
        sl = pl.ds(q * L, L)
        rows[r, sl] = rows[r, sl] * ws

  def g_start(j, rows, sem):
    pltpu.async_copy(h_hbm.at[src_v.at[j, 0]], rows, sem)

  def g_wait(j, rows, sem):
    pltpu.make_async_copy(h_hbm.at[src_v.at[j, 0]], rows, sem).wait()

  def s_start(j, rows, sem):
    pltpu.async_copy(rows, u_sh.at[dst_v.at[j, 0]], sem, add=True)

  def s_wait(j, rows, sem):
    pltpu.make_async_copy(rows, u_sh.at[dst_v.at[j, 0]], sem).wait()

  def den_scatter(j, wblk):
    pltpu.sync_copy(wblk, den_sh.at[dst_v.at[j, 0]], add=True)

  g_start(0, rows0_v, semg0)

  def body(i, carry):
    j0 = 2 * i
    j1 = j0 + 1
    compute_w(j0, wblk0_v)

    @pl.when(i > 0)
    def _():
      s_wait(j1 - 2, rows1_v, sems1)

    g_wait(j0, rows0_v, semg0)
    g_start(j1, rows1_v, semg1)
    scale(rows0_v, wblk0_v)
    s_start(j0, rows0_v, sems0)
    den_scatter(j0, wblk0_v)
    compute_w(j1, wblk1_v)
    s_wait(j0, rows0_v, sems0)
    g_wait(j1, rows1_v, semg1)
    g_start(j0 + 2, rows0_v, semg0)
    scale(rows1_v, wblk1_v)
    s_start(j1, rows1_v, sems1)
    den_scatter(j1, wblk1_v)
    return carry

  lax.fori_loop(0, (BPT - 1) // 2, body, 0)

  jt = BPT - 1
  compute_w(jt, wblk0_v)
  s_wait(jt - 2, rows1_v, sems1)
  g_wait(jt, rows0_v, semg0)
  scale(rows0_v, wblk0_v)
  s_start(jt, rows0_v, sems0)
  den_scatter(jt, wblk0_v)
  s_wait(jt, rows0_v, sems0)
  plsc.subcore_barrier()

  pltpu.sync_copy(u_sh.at[pl.ds(node0, NPT)], u_hbm.at[c, pl.ds(node0, NPT)])

  @pl.when(s == 0)
  def _():
    pltpu.sync_copy(den_sh, den_hbm.at[c])


@functools.partial(jax.jit, static_argnames=())
def _sc_conv(h, al, ar, src2, dst2, zu, zd, cs):
  mesh = plsc.VectorSubcoreMesh(
      core_axis_name="c", subcore_axis_name="s", num_cores=NC, num_subcores=NS
  )
  f32 = jnp.float32
  run = pl.kernel(
      _sc_body,
      out_type=(
          jax.ShapeDtypeStruct((NC, N2, D), f32),
          jax.ShapeDtypeStruct((NC, N2), f32),
      ),
      mesh=mesh,
      compiler_params=pltpu.CompilerParams(
          needs_layout_passes=False, use_tc_tiling_on_sc=False),
      scratch_types=(
          pltpu.VMEM((N,), f32),
          pltpu.VMEM((N,), f32),
          pltpu.VMEM((BPT, 1, RB), jnp.int32),
          pltpu.VMEM((BPT, 1, RB), jnp.int32),
          pltpu.VMEM((RB,), f32),
          pltpu.VMEM((RB,), f32),
          pltpu.VMEM((RB, D), f32),
          pltpu.VMEM((RB, D), f32),
          pltpu.VMEM((RB, L), jnp.int32),
          pltpu.VMEM_SHARED((N2, D), f32),
          pltpu.VMEM_SHARED((N2,), f32),
          pltpu.SemaphoreType.DMA,
          pltpu.SemaphoreType.DMA,
          pltpu.SemaphoreType.DMA,
          pltpu.SemaphoreType.DMA,
      ),
  )
  return run(h, al, ar, src2, dst2, zu, zd, cs)


def _tc_layer0_body(x_ref, w_ref, asrc_ref, adst_ref, h_ref, al_ref, ar_ref):
  h = jnp.dot(x_ref[...], w_ref[...], preferred_element_type=jnp.float32)
  h_ref[...] = h
  al_ref[...] = jnp.sum(h * asrc_ref[...], axis=1, keepdims=True)
  ar_ref[...] = jnp.sum(h * adst_ref[...], axis=1, keepdims=True)


def _tc_layer0(x, w, asrc, adst):
  f32 = jnp.float32
  return pl.pallas_call(
      _tc_layer0_body,
      out_shape=(
          jax.ShapeDtypeStruct((N, D), f32),
          jax.ShapeDtypeStruct((N, 1), f32),
          jax.ShapeDtypeStruct((N, 1), f32),
      ),
  )(x, w, asrc, adst)


def _combine(u_ref, den_ref, b_ref, rows=None):
  sl = slice(None) if rows is None else slice(0, rows)
  den = den_ref[0, sl] + den_ref[1, sl] + jnp.float32(_EPS)
  out = (u_ref[0, sl] + u_ref[1, sl]) / den + b_ref[...]
  return jnp.maximum(out, jnp.float32(0.0))

BLK = 2000


def _tc_mid_body(ua_ref, da_ref, ub_ref, db_ref, b_ref, wt_ref, wb_ref,
                 asrc_ref, adst_ref, h_ref, al_ref, ar_ref):
  ra = _combine(ua_ref, da_ref, b_ref)
  rb = _combine(ub_ref, db_ref, b_ref)
  h = (jnp.dot(ra, wt_ref[...], preferred_element_type=jnp.float32)
       + jnp.dot(rb, wb_ref[...], preferred_element_type=jnp.float32))
  h_ref[...] = h
  al_ref[...] = jnp.sum(h * asrc_ref[...], axis=1, keepdims=True)
  ar_ref[...] = jnp.sum(h * adst_ref[...], axis=1, keepdims=True)


def _tc_mid(ua, da, ub, db, b, wt, wb, asrc, adst):
  da = da.reshape(NC, N2, 1)
  db = db.reshape(NC, N2, 1)
  f32 = jnp.float32
  ubs = pl.BlockSpec((NC, BLK, D), lambda i: (0, i, 0))
  dbs = pl.BlockSpec((NC, BLK, 1), lambda i: (0, i, 0))
  full2 = lambda shape: pl.BlockSpec(shape, lambda i: (0, 0))
  return pl.pallas_call(
      _tc_mid_body,
      grid=(N // BLK,),
      in_specs=[ubs, dbs, ubs, dbs,
                full2((1, D)), full2((D, D)), full2((D, D)),
                full2((1, D)), full2((1, D))],
      out_specs=(
          pl.BlockSpec((BLK, D), lambda i: (i, 0)),
          pl.BlockSpec((BLK, 1), lambda i: (i, 0)),
          pl.BlockSpec((BLK, 1), lambda i: (i, 0)),
      ),
      out_shape=(
          jax.ShapeDtypeStruct((N, D), f32),
          jax.ShapeDtypeStruct((N, 1), f32),
          jax.ShapeDtypeStruct((N, 1), f32),
      ),
  )(ua, da, ub, db, b, wt, wb, asrc, adst)


def _tc_final_body(ua_ref, da_ref, ub_ref, db_ref, b_ref, wft_ref, wfb_ref,
                   bf_ref, batch_ref, y_ref):
  ra = _combine(ua_ref, da_ref, b_ref, rows=N)
  rb = _combine(ub_ref, db_ref, b_ref, rows=N)
  z = (jnp.dot(ra, wft_ref[...], preferred_element_type=jnp.float32)
       + jnp.dot(rb, wfb_ref[...], preferred_element_type=jnp.float32))
  gid = lax.broadcasted_iota(jnp.int32, (G, N), 0)
  m = (gid == batch_ref[...]).astype(jnp.float32)
  y = jnp.dot(m, z, preferred_element_type=jnp.float32)
  y_ref[...] = y + bf_ref[...]


def _tc_final(ua, da, ub, db, b, wft, wfb, bf, batch):
  da = da.reshape(NC, N2, 1)
  db = db.reshape(NC, N2, 1)
  return pl.pallas_call(
      _tc_final_body,
      out_shape=jax.ShapeDtypeStruct((G, 1), jnp.float32),
  )(ua, da, ub, db, b, wft, wfb, bf, batch)


def kernel(x, edge_index, dense_edge_idx, batch,
           W0, a_src0, a_dst0, b0,
           W1, a_src1, a_dst1, b1,
           W2, a_src2, a_dst2, b2,
           Wf, bf):
  f32 = jnp.float32
  srcA = edge_index[0].reshape(ROWS, 1, RB)
  dstA = edge_index[1].reshape(ROWS, 1, RB)
  srcB = dense_edge_idx[0].reshape(ROWS, 1, RB)
  dstB = dense_edge_idx[1].reshape(ROWS, 1, RB)
  zu = jnp.zeros((N2, D), f32)
  zd = jnp.zeros((N2,), f32)
  cs = jnp.broadcast_to(jnp.arange(RB, dtype=jnp.int32)[:, None], (RB, L))
  cs = jnp.asarray(cs)
  batch2 = batch.reshape(1, N)

  params = [
      (W0, a_src0, a_dst0, b0),
      (W1, a_src1, a_dst1, b1),
      (W2, a_src2, a_dst2, b2),
  ]

  h, al, ar = _tc_layer0(x, W0, a_src0.reshape(1, D), a_dst0.reshape(1, D))
  for l in (1, 2):
    ua, da = _sc_conv(h, al.reshape(N), ar.reshape(N), srcA, dstA, zu, zd, cs)
    ub, db = _sc_conv(h, al.reshape(N), ar.reshape(N), srcB, dstB, zu, zd, cs)
    W, asrc, adst, _ = params[l]
    b_prev = params[l - 1][3]
    h, al, ar = _tc_mid(
        ua, da, ub, db, b_prev.reshape(1, D),
        W[:D], W[D:], asrc.reshape(1, D), adst.reshape(1, D))

  ua, da = _sc_conv(h, al.reshape(N), ar.reshape(N), srcA, dstA, zu, zd, cs)
  ub, db = _sc_conv(h, al.reshape(N), ar.reshape(N), srcB, dstB, zu, zd, cs)
  y = _tc_final(
      ua, da, ub, db, b2.reshape(1, D),
      Wf[:D], Wf[D:], bf.reshape(1, 1), batch2)
  return y.reshape(G)


# speedup vs baseline: 44.8112x; 1.0190x over previous
import functools

import jax
import jax.numpy as jnp
from jax import lax
from jax.experimental import pallas as pl
from jax.experimental.pallas import tpu as pltpu
from jax.experimental.pallas import tpu_sc as plsc

N = 10000
E = 320000
G = 64
D = 64
F = 128
NC = 2
NS = 16
L = 16
RB = 80
EPT = E // (NC * NS)
BPT = EPT // RB
ROWS = E // RB
N2 = 10240
NPT = N2 // NS

_EPS = 1e-16


def _sc_body(h_hbm, al_hbm, ar_hbm, src_hbm, dst_hbm, zu_hbm, zd_hbm, cs_hbm,
             u_hbm, den_hbm,
             al_v, ar_v, src_v, dst_v, wblk0_v, wblk1_v,
             grow0_v, grow1_v, srow0_v, srow1_v,
             cs_v, u_sh, den_sh,
             semg0, semg1, sems0, sems1, semd0, semd1):
  c = lax.axis_index("c")
  s = lax.axis_index("s")
  tile = c * NS + s
  row0 = tile * BPT
  node0 = s * NPT

  pltpu.sync_copy(al_hbm, al_v)
  pltpu.sync_copy(ar_hbm, ar_v)
  pltpu.sync_copy(src_hbm.at[pl.ds(row0, BPT)], src_v)
  pltpu.sync_copy(dst_hbm.at[pl.ds(row0, BPT)], dst_v)
  pltpu.sync_copy(zu_hbm.at[pl.ds(node0, NPT)], u_sh.at[pl.ds(node0, NPT)])
  pltpu.sync_copy(cs_hbm, cs_v)

  @pl.when(s == 0)
  def _():
    pltpu.sync_copy(zd_hbm, den_sh)

  plsc.subcore_barrier()

  def compute_w(j, wblk):
    for g in range(RB // L):
      sl = pl.ds(g * L, L)
      isrc = src_v[j, 0, sl]
      idst = dst_v[j, 0, sl]
      a = plsc.load_gather(al_v, [isrc])
      r = plsc.load_gather(ar_v, [idst])
      e = a + r
      e = jnp.where(e >= 0, e, e * jnp.float32(0.2))
      wblk[sl] = jnp.exp(e)

  def scale_into(gbuf, sbuf, wblk):
    for r in range(RB):
      ws = plsc.load_gather(wblk, [cs_v[r]])
      for q in range(D // L):
        sl = pl.ds(q * L, L)
        sbuf[r, sl] = gbuf[r, sl] * ws

  def g_start(j, gbuf, sem):
    pltpu.async_copy(h_hbm.at[src_v.at[j, 0]], gbuf, sem)

  def g_wait(j, gbuf, sem):
    pltpu.make_async_copy(h_hbm.at[src_v.at[j, 0]], gbuf, sem).wait()

  def s_start(j, sbuf, sem):
    pltpu.async_copy(sbuf, u_sh.at[dst_v.at[j, 0]], sem, add=True)

  def s_wait(j, sbuf, sem):
    pltpu.make_async_copy(sbuf, u_sh.at[dst_v.at[j, 0]], sem).wait()

  def d_start(j, wblk, sem):
    pltpu.async_copy(wblk, den_sh.at[dst_v.at[j, 0]], sem, add=True)

  def d_wait(j, wblk, sem):
    pltpu.make_async_copy(wblk, den_sh.at[dst_v.at[j, 0]], sem).wait()

  gbufs = (grow0_v, grow1_v)
  sbufs = (srow0_v, srow1_v)
  wblks = (wblk0_v, wblk1_v)
  semg = (semg0, semg1)
  sems = (sems0, sems1)
  semd = (semd0, semd1)

  def slot(j, p, first):
    if not first:
      d_wait(j - 2, wblks[p], semd[p])
    compute_w(j, wblks[p])
    g_wait(j, gbufs[p], semg[p])
    if not first:
      s_wait(j - 2, sbufs[p], sems[p])
    scale_into(gbufs[p], sbufs[p], wblks[p])
    nxt = j + 2

    @pl.when(nxt < BPT)
    def _():
      g_start(nxt, gbufs[p], semg[p])

    s_start(j, sbufs[p], sems[p])
    d_start(j, wblks[p], semd[p])

  g_start(0, gbufs[0], semg[0])
  g_start(1, gbufs[1], semg[1])

  def first_body(_, carry):
    slot(0, 0, True)
    slot(1, 1, True)
    return carry

  lax.fori_loop(0, 1, first_body, 0)

  def body(i, carry):
    j0 = 2 * (i + 1)
    slot(j0, 0, False)
    slot(j0 + 1, 1, False)
    return carry

  lax.fori_loop(0, (BPT - 3) // 2, body, 0)

  jt = BPT - 1
  slot(jt, 0, False)
  s_wait(jt - 1, sbufs[1], sems[1])
  d_wait(jt - 1, wblks[1], semd[1])
  s_wait(jt, sbufs[0], sems[0])
  d_wait(jt, wblks[0], semd[0])
  plsc.subcore_barrier()

  pltpu.sync_copy(u_sh.at[pl.ds(node0, NPT)], u_hbm.at[c, pl.ds(node0, NPT)])

  @pl.when(s == 0)
  def _():
    pltpu.sync_copy(den_sh, den_hbm.at[c])


@functools.partial(jax.jit, static_argnames=())
def _sc_conv(h, al, ar, src2, dst2, zu, zd, cs):
  mesh = plsc.VectorSubcoreMesh(
      core_axis_name="c", subcore_axis_name="s", num_cores=NC, num_subcores=NS
  )
  f32 = jnp.float32
  run = pl.kernel(
      _sc_body,
      out_type=(
          jax.ShapeDtypeStruct((NC, N2, D), f32),
          jax.ShapeDtypeStruct((NC, N2), f32),
      ),
      mesh=mesh,
      compiler_params=pltpu.CompilerParams(
          needs_layout_passes=False, use_tc_tiling_on_sc=False),
      scratch_types=(
          pltpu.VMEM((N,), f32),
          pltpu.VMEM((N,), f32),
          pltpu.VMEM((BPT, 1, RB), jnp.int32),
          pltpu.VMEM((BPT, 1, RB), jnp.int32),
          pltpu.VMEM((RB,), f32),
          pltpu.VMEM((RB,), f32),
          pltpu.VMEM((RB, D), f32),
          pltpu.VMEM((RB, D), f32),
          pltpu.VMEM((RB, D), f32),
          pltpu.VMEM((RB, D), f32),
          pltpu.VMEM((RB, L), jnp.int32),
          pltpu.VMEM_SHARED((N2, D), f32),
          pltpu.VMEM_SHARED((N2,), f32),
          pltpu.SemaphoreType.DMA,
          pltpu.SemaphoreType.DMA,
          pltpu.SemaphoreType.DMA,
          pltpu.SemaphoreType.DMA,
          pltpu.SemaphoreType.DMA,
          pltpu.SemaphoreType.DMA,
      ),
  )
  return run(h, al, ar, src2, dst2, zu, zd, cs)


def _tc_layer0_body(x_ref, w_ref, asrc_ref, adst_ref, h_ref, al_ref, ar_ref):
  h = jnp.dot(x_ref[...], w_ref[...], preferred_element_type=jnp.float32)
  h_ref[...] = h
  al_ref[...] = jnp.sum(h * asrc_ref[...], axis=1, keepdims=True)
  ar_ref[...] = jnp.sum(h * adst_ref[...], axis=1, keepdims=True)


def _tc_layer0(x, w, asrc, adst):
  f32 = jnp.float32
  return pl.pallas_call(
      _tc_layer0_body,
      out_shape=(
          jax.ShapeDtypeStruct((N, D), f32),
          jax.ShapeDtypeStruct((N, 1), f32),
          jax.ShapeDtypeStruct((N, 1), f32),
      ),
  )(x, w, asrc, adst)


def _combine(u_ref, den_ref, b_ref, rows=None):
  sl = slice(None) if rows is None else slice(0, rows)
  den = den_ref[0, sl] + den_ref[1, sl] + jnp.float32(_EPS)
  out = (u_ref[0, sl] + u_ref[1, sl]) / den + b_ref[...]
  return jnp.maximum(out, jnp.float32(0.0))

BLK = 2000


def _tc_mid_body(ua_ref, da_ref, ub_ref, db_ref, b_ref, wt_ref, wb_ref,
                 asrc_ref, adst_ref, h_ref, al_ref, ar_ref):
  ra = _combine(ua_ref, da_ref, b_ref)
  rb = _combine(ub_ref, db_ref, b_ref)
  h = (jnp.dot(ra, wt_ref[...], preferred_element_type=jnp.float32)
       + jnp.dot(rb, wb_ref[...], preferred_element_type=jnp.float32))
  h_ref[...] = h
  al_ref[...] = jnp.sum(h * asrc_ref[...], axis=1, keepdims=True)
  ar_ref[...] = jnp.sum(h * adst_ref[...], axis=1, keepdims=True)


def _tc_mid(ua, da, ub, db, b, wt, wb, asrc, adst):
  da = da.reshape(NC, N2, 1)
  db = db.reshape(NC, N2, 1)
  f32 = jnp.float32
  ubs = pl.BlockSpec((NC, BLK, D), lambda i: (0, i, 0))
  dbs = pl.BlockSpec((NC, BLK, 1), lambda i: (0, i, 0))
  full2 = lambda shape: pl.BlockSpec(shape, lambda i: (0, 0))
  return pl.pallas_call(
      _tc_mid_body,
      grid=(N // BLK,),
      in_specs=[ubs, dbs, ubs, dbs,
                full2((1, D)), full2((D, D)), full2((D, D)),
                full2((1, D)), full2((1, D))],
      out_specs=(
          pl.BlockSpec((BLK, D), lambda i: (i, 0)),
          pl.BlockSpec((BLK, 1), lambda i: (i, 0)),
          pl.BlockSpec((BLK, 1), lambda i: (i, 0)),
      ),
      out_shape=(
          jax.ShapeDtypeStruct((N, D), f32),
          jax.ShapeDtypeStruct((N, 1), f32),
          jax.ShapeDtypeStruct((N, 1), f32),
      ),
  )(ua, da, ub, db, b, wt, wb, asrc, adst)


def _tc_final_body(ua_ref, da_ref, ub_ref, db_ref, b_ref, wft_ref, wfb_ref,
                   bf_ref, batch_ref, y_ref):
  ra = _combine(ua_ref, da_ref, b_ref, rows=N)
  rb = _combine(ub_ref, db_ref, b_ref, rows=N)
  z = (jnp.dot(ra, wft_ref[...], preferred_element_type=jnp.float32)
       + jnp.dot(rb, wfb_ref[...], preferred_element_type=jnp.float32))
  gid = lax.broadcasted_iota(jnp.int32, (G, N), 0)
  m = (gid == batch_ref[...]).astype(jnp.float32)
  y = jnp.dot(m, z, preferred_element_type=jnp.float32)
  y_ref[...] = y + bf_ref[...]


def _tc_final(ua, da, ub, db, b, wft, wfb, bf, batch):
  da = da.reshape(NC, N2, 1)
  db = db.reshape(NC, N2, 1)
  return pl.pallas_call(
      _tc_final_body,
      out_shape=jax.ShapeDtypeStruct((G, 1), jnp.float32),
  )(ua, da, ub, db, b, wft, wfb, bf, batch)


def kernel(x, edge_index, dense_edge_idx, batch,
           W0, a_src0, a_dst0, b0,
           W1, a_src1, a_dst1, b1,
           W2, a_src2, a_dst2, b2,
           Wf, bf):
  f32 = jnp.float32
  srcA = edge_index[0].reshape(ROWS, 1, RB)
  dstA = edge_index[1].reshape(ROWS, 1, RB)
  srcB = dense_edge_idx[0].reshape(ROWS, 1, RB)
  dstB = dense_edge_idx[1].reshape(ROWS, 1, RB)
  zu = jnp.zeros((N2, D), f32)
  zd = jnp.zeros((N2,), f32)
  cs = jnp.broadcast_to(jnp.arange(RB, dtype=jnp.int32)[:, None], (RB, L))
  cs = jnp.asarray(cs)
  batch2 = batch.reshape(1, N)

  params = [
      (W0, a_src0, a_dst0, b0),
      (W1, a_src1, a_dst1, b1),
      (W2, a_src2, a_dst2, b2),
  ]

  h, al, ar = _tc_layer0(x, W0, a_src0.reshape(1, D), a_dst0.reshape(1, D))
  for l in (1, 2):
    ua, da = _sc_conv(h, al.reshape(N), ar.reshape(N), srcA, dstA, zu, zd, cs)
    ub, db = _sc_conv(h, al.reshape(N), ar.reshape(N), srcB, dstB, zu, zd, cs)
    W, asrc, adst, _ = params[l]
    b_prev = params[l - 1][3]
    h, al, ar = _tc_mid(
        ua, da, ub, db, b_prev.reshape(1, D),
        W[:D], W[D:], asrc.reshape(1, D), adst.reshape(1, D))

  ua, da = _sc_conv(h, al.reshape(N), ar.reshape(N), srcA, dstA, zu, zd, cs)
  ub, db = _sc_conv(h, al.reshape(N), ar.reshape(N), srcB, dstB, zu, zd, cs)
  y = _tc_final(
      ua, da, ub, db, b2.reshape(1, D),
      Wf[:D], Wf[D:], bf.reshape(1, 1), batch2)
  return y.reshape(G)


# speedup vs baseline: 75.0971x; 1.6759x over previous
import functools

import jax
import jax.numpy as jnp
from jax import lax
from jax.experimental import pallas as pl
from jax.experimental.pallas import tpu as pltpu
from jax.experimental.pallas import tpu_sc as plsc

N = 10000
E = 320000
G = 64
D = 64
F = 128
NC = 2
NS = 16
L = 16
RB = 80
EPT = E // (NC * NS)
BPT = EPT // RB
ROWS = E // RB
N2 = 10240
NPT = N2 // NS

_EPS = 1e-16


def _sc_body(h_hbm, al_hbm, ar_hbm, src_hbm, dst_hbm, zu_hbm, zd_hbm, cs_hbm,
             u_hbm, den_hbm,
             al_v, ar_v, src_v, dst_v, wblk0_v, wblk1_v,
             grow0_v, grow1_v, srow0_v, srow1_v,
             cs_v, u_sh, den_sh,
             semg0, semg1, sems0, sems1, semd0, semd1):
  c = lax.axis_index("c")
  s = lax.axis_index("s")
  tile = c * NS + s
  row0 = tile * BPT
  node0 = s * NPT

  pltpu.sync_copy(al_hbm, al_v)
  pltpu.sync_copy(ar_hbm, ar_v)
  pltpu.sync_copy(src_hbm.at[pl.ds(row0, BPT)], src_v)
  pltpu.sync_copy(dst_hbm.at[pl.ds(row0, BPT)], dst_v)
  pltpu.sync_copy(zu_hbm.at[pl.ds(node0, NPT)], u_sh.at[pl.ds(node0, NPT)])
  pltpu.sync_copy(cs_hbm, cs_v)

  @pl.when(s == 0)
  def _():
    pltpu.sync_copy(zd_hbm, den_sh)

  plsc.subcore_barrier()

  def compute_w(j, wblk):
    for g in range(RB // L):
      sl = pl.ds(g * L, L)
      isrc = src_v[j, 0, sl]
      idst = dst_v[j, 0, sl]
      a = plsc.load_gather(al_v, [isrc])
      r = plsc.load_gather(ar_v, [idst])
      e = a + r
      e = jnp.where(e >= 0, e, e * jnp.float32(0.2))
      wblk[sl] = jnp.exp(e)

  def scale_into(gbuf, sbuf, wblk):
    for g in range(RB // L):
      wv = wblk[pl.ds(g * L, L)]
      for t in range(L):
        r = g * L + t
        ws = jnp.broadcast_to(wv[t], (L,))
        for q in range(D // L):
          sl = pl.ds(q * L, L)
          sbuf[r, sl] = gbuf[r, sl] * ws

  def g_start(j, gbuf, sem):
    pltpu.async_copy(h_hbm.at[src_v.at[j, 0]], gbuf, sem)

  def g_wait(j, gbuf, sem):
    pltpu.make_async_copy(h_hbm.at[src_v.at[j, 0]], gbuf, sem).wait()

  def s_start(j, sbuf, sem):
    pltpu.async_copy(sbuf, u_sh.at[dst_v.at[j, 0]], sem, add=True)

  def s_wait(j, sbuf, sem):
    pltpu.make_async_copy(sbuf, u_sh.at[dst_v.at[j, 0]], sem).wait()

  def d_start(j, wblk, sem):
    pltpu.async_copy(wblk, den_sh.at[dst_v.at[j, 0]], sem, add=True)

  def d_wait(j, wblk, sem):
    pltpu.make_async_copy(wblk, den_sh.at[dst_v.at[j, 0]], sem).wait()

  gbufs = (grow0_v, grow1_v)
  sbufs = (srow0_v, srow1_v)
  wblks = (wblk0_v, wblk1_v)
  semg = (semg0, semg1)
  sems = (sems0, sems1)
  semd = (semd0, semd1)

  def slot(j, p, first):
    if not first:
      d_wait(j - 2, wblks[p], semd[p])
    compute_w(j, wblks[p])
    g_wait(j, gbufs[p], semg[p])
    if not first:
      s_wait(j - 2, sbufs[p], sems[p])
    scale_into(gbufs[p], sbufs[p], wblks[p])
    nxt = j + 2

    @pl.when(nxt < BPT)
    def _():
      g_start(nxt, gbufs[p], semg[p])

    s_start(j, sbufs[p], sems[p])
    d_start(j, wblks[p], semd[p])

  g_start(0, gbufs[0], semg[0])
  g_start(1, gbufs[1], semg[1])

  def first_body(_, carry):
    slot(0, 0, True)
    slot(1, 1, True)
    return carry

  lax.fori_loop(0, 1, first_body, 0)

  def body(i, carry):
    j0 = 2 * (i + 1)
    slot(j0, 0, False)
    slot(j0 + 1, 1, False)
    return carry

  lax.fori_loop(0, (BPT - 3) // 2, body, 0)

  jt = BPT - 1
  slot(jt, 0, False)
  s_wait(jt - 1, sbufs[1], sems[1])
  d_wait(jt - 1, wblks[1], semd[1])
  s_wait(jt, sbufs[0], sems[0])
  d_wait(jt, wblks[0], semd[0])
  plsc.subcore_barrier()

  pltpu.sync_copy(u_sh.at[pl.ds(node0, NPT)], u_hbm.at[c, pl.ds(node0, NPT)])

  @pl.when(s == 0)
  def _():
    pltpu.sync_copy(den_sh, den_hbm.at[c])


@functools.partial(jax.jit, static_argnames=())
def _sc_conv(h, al, ar, src2, dst2, zu, zd, cs):
  mesh = plsc.VectorSubcoreMesh(
      core_axis_name="c", subcore_axis_name="s", num_cores=NC, num_subcores=NS
  )
  f32 = jnp.float32
  run = pl.kernel(
      _sc_body,
      out_type=(
          jax.ShapeDtypeStruct((NC, N2, D), f32),
          jax.ShapeDtypeStruct((NC, N2), f32),
      ),
      mesh=mesh,
      compiler_params=pltpu.CompilerParams(
          needs_layout_passes=False, use_tc_tiling_on_sc=False),
      scratch_types=(
          pltpu.VMEM((N,), f32),
          pltpu.VMEM((N,), f32),
          pltpu.VMEM((BPT, 1, RB), jnp.int32),
          pltpu.VMEM((BPT, 1, RB), jnp.int32),
          pltpu.VMEM((RB,), f32),
          pltpu.VMEM((RB,), f32),
          pltpu.VMEM((RB, D), f32),
          pltpu.VMEM((RB, D), f32),
          pltpu.VMEM((RB, D), f32),
          pltpu.VMEM((RB, D), f32),
          pltpu.VMEM((RB, L), jnp.int32),
          pltpu.VMEM_SHARED((N2, D), f32),
          pltpu.VMEM_SHARED((N2,), f32),
          pltpu.SemaphoreType.DMA,
          pltpu.SemaphoreType.DMA,
          pltpu.SemaphoreType.DMA,
          pltpu.SemaphoreType.DMA,
          pltpu.SemaphoreType.DMA,
          pltpu.SemaphoreType.DMA,
      ),
  )
  return run(h, al, ar, src2, dst2, zu, zd, cs)


def _tc_layer0_body(x_ref, w_ref, asrc_ref, adst_ref, h_ref, al_ref, ar_ref):
  h = jnp.dot(x_ref[...], w_ref[...], preferred_element_type=jnp.float32)
  h_ref[...] = h
  al_ref[...] = jnp.sum(h * asrc_ref[...], axis=1, keepdims=True)
  ar_ref[...] = jnp.sum(h * adst_ref[...], axis=1, keepdims=True)


def _tc_layer0(x, w, asrc, adst):
  f32 = jnp.float32
  return pl.pallas_call(
      _tc_layer0_body,
      out_shape=(
          jax.ShapeDtypeStruct((N, D), f32),
          jax.ShapeDtypeStruct((N, 1), f32),
          jax.ShapeDtypeStruct((N, 1), f32),
      ),
  )(x, w, asrc, adst)


def _combine(u_ref, den_ref, b_ref, rows=None):
  sl = slice(None) if rows is None else slice(0, rows)
  den = den_ref[0, sl] + den_ref[1, sl] + jnp.float32(_EPS)
  out = (u_ref[0, sl] + u_ref[1, sl]) / den + b_ref[...]
  return jnp.maximum(out, jnp.float32(0.0))

BLK = 2000


def _tc_mid_body(ua_ref, da_ref, ub_ref, db_ref, b_ref, wt_ref, wb_ref,
                 asrc_ref, adst_ref, h_ref, al_ref, ar_ref):
  ra = _combine(ua_ref, da_ref, b_ref)
  rb = _combine(ub_ref, db_ref, b_ref)
  h = (jnp.dot(ra, wt_ref[...], preferred_element_type=jnp.float32)
       + jnp.dot(rb, wb_ref[...], preferred_element_type=jnp.float32))
  h_ref[...] = h
  al_ref[...] = jnp.sum(h * asrc_ref[...], axis=1, keepdims=True)
  ar_ref[...] = jnp.sum(h * adst_ref[...], axis=1, keepdims=True)


def _tc_mid(ua, da, ub, db, b, wt, wb, asrc, adst):
  da = da.reshape(NC, N2, 1)
  db = db.reshape(NC, N2, 1)
  f32 = jnp.float32
  ubs = pl.BlockSpec((NC, BLK, D), lambda i: (0, i, 0))
  dbs = pl.BlockSpec((NC, BLK, 1), lambda i: (0, i, 0))
  full2 = lambda shape: pl.BlockSpec(shape, lambda i: (0, 0))
  return pl.pallas_call(
      _tc_mid_body,
      grid=(N // BLK,),
      in_specs=[ubs, dbs, ubs, dbs,
                full2((1, D)), full2((D, D)), full2((D, D)),
                full2((1, D)), full2((1, D))],
      out_specs=(
          pl.BlockSpec((BLK, D), lambda i: (i, 0)),
          pl.BlockSpec((BLK, 1), lambda i: (i, 0)),
          pl.BlockSpec((BLK, 1), lambda i: (i, 0)),
      ),
      out_shape=(
          jax.ShapeDtypeStruct((N, D), f32),
          jax.ShapeDtypeStruct((N, 1), f32),
          jax.ShapeDtypeStruct((N, 1), f32),
      ),
  )(ua, da, ub, db, b, wt, wb, asrc, adst)


def _tc_final_body(ua_ref, da_ref, ub_ref, db_ref, b_ref, wft_ref, wfb_ref,
                   bf_ref, batch_ref, y_ref):
  ra = _combine(ua_ref, da_ref, b_ref, rows=N)
  rb = _combine(ub_ref, db_ref, b_ref, rows=N)
  z = (jnp.dot(ra, wft_ref[...], preferred_element_type=jnp.float32)
       + jnp.dot(rb, wfb_ref[...], preferred_element_type=jnp.float32))
  gid = lax.broadcasted_iota(jnp.int32, (G, N), 0)
  m = (gid == batch_ref[...]).astype(jnp.float32)
  y = jnp.dot(m, z, preferred_element_type=jnp.float32)
  y_ref[...] = y + bf_ref[...]


def _tc_final(ua, da, ub, db, b, wft, wfb, bf, batch):
  da = da.reshape(NC, N2, 1)
  db = db.reshape(NC, N2, 1)
  return pl.pallas_call(
      _tc_final_body,
      out_shape=jax.ShapeDtypeStruct((G, 1), jnp.float32),
  )(ua, da, ub, db, b, wft, wfb, bf, batch)


def kernel(x, edge_index, dense_edge_idx, batch,
           W0, a_src0, a_dst0, b0,
           W1, a_src1, a_dst1, b1,
           W2, a_src2, a_dst2, b2,
           Wf, bf):
  f32 = jnp.float32
  srcA = edge_index[0].reshape(ROWS, 1, RB)
  dstA = edge_index[1].reshape(ROWS, 1, RB)
  srcB = dense_edge_idx[0].reshape(ROWS, 1, RB)
  dstB = dense_edge_idx[1].reshape(ROWS, 1, RB)
  zu = jnp.zeros((N2, D), f32)
  zd = jnp.zeros((N2,), f32)
  cs = jnp.broadcast_to(jnp.arange(RB, dtype=jnp.int32)[:, None], (RB, L))
  cs = jnp.asarray(cs)
  batch2 = batch.reshape(1, N)

  params = [
      (W0, a_src0, a_dst0, b0),
      (W1, a_src1, a_dst1, b1),
      (W2, a_src2, a_dst2, b2),
  ]

  h, al, ar = _tc_layer0(x, W0, a_src0.reshape(1, D), a_dst0.reshape(1, D))
  for l in (1, 2):
    ua, da = _sc_conv(h, al.reshape(N), ar.reshape(N), srcA, dstA, zu, zd, cs)
    ub, db = _sc_conv(h, al.reshape(N), ar.reshape(N), srcB, dstB, zu, zd, cs)
    W, asrc, adst, _ = params[l]
    b_prev = params[l - 1][3]
    h, al, ar = _tc_mid(
        ua, da, ub, db, b_prev.reshape(1, D),
        W[:D], W[D:], asrc.reshape(1, D), adst.reshape(1, D))

  ua, da = _sc_conv(h, al.reshape(N), ar.reshape(N), srcA, dstA, zu, zd, cs)
  ub, db = _sc_conv(h, al.reshape(N), ar.reshape(N), srcB, dstB, zu, zd, cs)
  y = _tc_final(
      ua, da, ub, db, b2.reshape(1, D),
      Wf[:D], Wf[D:], bf.reshape(1, 1), batch2)
  return y.reshape(G)
